# Initial kernel scaffold; baseline (speedup 1.0000x reference)
import jax
import jax.numpy as jnp
from jax.experimental import pallas as pl


def kernel(pred_logits, pred_boxes, tgt_boxes, tgt_labels, src_idx):
    raise NotImplementedError("write your pallas kernel here")


# speedup vs baseline: 2.8333x; 2.8333x over previous
import functools

import jax
import jax.numpy as jnp
from jax import lax
from jax.experimental import pallas as pl
from jax.experimental.pallas import tpu as pltpu
from jax.experimental.pallas import tpu_sc as plsc

_B, _NQ, _C, _NG = 32, 300, 80, 50
_NGP = 64
_NQP = 304
_ALPHA = 0.75
_W_VFL, _W_BBOX, _W_GIOU = 1.0, 5.0, 2.0
_L = 16


def _sc_body(logits_hbm, boxes_hbm, tgtb_hbm, sidx_hbm, lab_hbm,
             iou_hbm, xl_hbm, l1_hbm, gt_hbm, mask_hbm,
             logits_v, boxes_v, tgtb_v, sidx_v, lab_v,
             iou_v, xl_v, l1_v, gt_v, mask_v):
    b = lax.axis_index("s") * 2 + lax.axis_index("c")
    pltpu.sync_copy(logits_hbm.at[b], logits_v)
    pltpu.sync_copy(boxes_hbm.at[b], boxes_v)
    pltpu.sync_copy(tgtb_hbm.at[b], tgtb_v)
    pltpu.sync_copy(sidx_hbm.at[b], sidx_v)
    pltpu.sync_copy(lab_hbm.at[b], lab_v)

    zeros = jnp.zeros((_L,), jnp.float32)
    ones = jnp.ones((_L,), jnp.float32)
    for i in range(_NQP // _L):
        mask_v[pl.ds(i * _L, _L)] = zeros

    lane = lax.iota(jnp.int32, _L)
    for g in range(_NGP // _L):
        qi = sidx_v[pl.ds(g * _L, _L)]
        li = lab_v[pl.ds(g * _L, _L)]
        gl = lane + (g * _L)
        valid = gl < _NG
        xl = plsc.load_gather(logits_v, [qi * _C + li])
        qb = qi * 4
        scx = plsc.load_gather(boxes_v, [qb])
        scy = plsc.load_gather(boxes_v, [qb + 1])
        sw = plsc.load_gather(boxes_v, [qb + 2])
        sh = plsc.load_gather(boxes_v, [qb + 3])
        tq = jnp.minimum(gl, _NG - 1) * 4
        tcx = plsc.load_gather(tgtb_v, [tq])
        tcy = plsc.load_gather(tgtb_v, [tq + 1])
        tw = plsc.load_gather(tgtb_v, [tq + 2])
        th = plsc.load_gather(tgtb_v, [tq + 3])
        sx1 = scx - 0.5 * sw
        sy1 = scy - 0.5 * sh
        sx2 = scx + 0.5 * sw
        sy2 = scy + 0.5 * sh
        tx1 = tcx - 0.5 * tw
        ty1 = tcy - 0.5 * th
        tx2 = tcx + 0.5 * tw
        ty2 = tcy + 0.5 * th
        a1 = (sx2 - sx1) * (sy2 - sy1)
        a2 = (tx2 - tx1) * (ty2 - ty1)
        iw = jnp.maximum(jnp.minimum(sx2, tx2) - jnp.maximum(sx1, tx1), 0.0)
        ih = jnp.maximum(jnp.minimum(sy2, ty2) - jnp.maximum(sy1, ty1), 0.0)
        inter = iw * ih
        union = a1 + a2 - inter
        iou = jnp.maximum(inter / union, 0.0)
        ew = jnp.maximum(sx2, tx2) - jnp.minimum(sx1, tx1)
        eh = jnp.maximum(sy2, ty2) - jnp.minimum(sy1, ty1)
        enc = ew * eh
        giou = iou - (enc - union) / enc
        l1 = (jnp.abs(scx - tcx) + jnp.abs(scy - tcy)
              + jnp.abs(sw - tw) + jnp.abs(sh - th))
        sl = pl.ds(g * _L, _L)
        iou_v[sl] = jnp.where(valid, iou, zeros)
        xl_v[sl] = jnp.where(valid, xl, zeros)
        l1_v[sl] = jnp.where(valid, l1, zeros)
        gt_v[sl] = jnp.where(valid, 1.0 - giou, zeros)
        plsc.store_scatter(mask_v, [qi], ones, mask=valid)

    pltpu.sync_copy(iou_v, iou_hbm.at[b])
    pltpu.sync_copy(xl_v, xl_hbm.at[b])
    pltpu.sync_copy(l1_v, l1_hbm.at[b])
    pltpu.sync_copy(gt_v, gt_hbm.at[b])
    pltpu.sync_copy(mask_v, mask_hbm.at[b])


@functools.cache
def _get_sc_call():
    return pl.kernel(
        _sc_body,
        out_type=(
            jax.ShapeDtypeStruct((_B, _NGP), jnp.float32),
            jax.ShapeDtypeStruct((_B, _NGP), jnp.float32),
            jax.ShapeDtypeStruct((_B, _NGP), jnp.float32),
            jax.ShapeDtypeStruct((_B, _NGP), jnp.float32),
            jax.ShapeDtypeStruct((_B, _NQP), jnp.float32),
        ),
        mesh=plsc.VectorSubcoreMesh(core_axis_name="c", subcore_axis_name="s"),
        compiler_params=pltpu.CompilerParams(needs_layout_passes=False),
        scratch_types=[
        pltpu.VMEM((_NQ * _C,), jnp.float32),
        pltpu.VMEM((_NQ * 4,), jnp.float32),
        pltpu.VMEM((_NG * 4,), jnp.float32),
        pltpu.VMEM((_NGP,), jnp.int32),
        pltpu.VMEM((_NGP,), jnp.int32),
        pltpu.VMEM((_NGP,), jnp.float32),
        pltpu.VMEM((_NGP,), jnp.float32),
        pltpu.VMEM((_NGP,), jnp.float32),
        pltpu.VMEM((_NGP,), jnp.float32),
        pltpu.VMEM((_NQP,), jnp.float32),
        ],
    )


def _tc_body(x_ref, mask_ref, iou_ref, xl_ref, l1_ref, gt_ref, out_ref):
    x = x_ref[...]
    t = jnp.exp(-jnp.abs(x))
    sig = jnp.where(x >= 0.0, 1.0, t) / (1.0 + t)
    sp = jnp.maximum(x, 0.0) + jnp.log1p(t)
    neg = _ALPHA * (sig * sig) * sp
    m = mask_ref[...][:, :_NQ]
    s_neg = jnp.sum(neg * (1.0 - m)[:, :, None])
    iou = iou_ref[...]
    xl = xl_ref[...]
    spl = jnp.maximum(xl, 0.0) + jnp.log1p(jnp.exp(-jnp.abs(xl)))
    pos = jnp.sum(iou * (spl - xl * iou))
    l1s = jnp.sum(l1_ref[...])
    gts = jnp.sum(gt_ref[...])
    nb = float(_B * _NG)
    out_ref[0, 0] = (_W_VFL * (s_neg + pos) + _W_BBOX * l1s + _W_GIOU * gts) / nb


_tc_call = pl.pallas_call(
    _tc_body,
    out_shape=jax.ShapeDtypeStruct((1, 1), jnp.float32),
    out_specs=pl.BlockSpec(memory_space=pltpu.SMEM),
)


def kernel(pred_logits, pred_boxes, tgt_boxes, tgt_labels, src_idx):
    logits_flat = pred_logits.reshape(_B, _NQ * _C)
    boxes_flat = pred_boxes.reshape(_B, _NQ * 4)
    tgtb_flat = tgt_boxes.reshape(_B, _NG * 4)
    padcfg = ((0, 0), (0, _NGP - _NG))
    sidx_p = jnp.pad(src_idx.astype(jnp.int32), padcfg)
    lab_p = jnp.pad(tgt_labels.astype(jnp.int32), padcfg)
    iou, xl, l1, gt, mask = _get_sc_call()(
        logits_flat, boxes_flat, tgtb_flat, sidx_p, lab_p)
    out = _tc_call(pred_logits, mask, iou, xl, l1, gt)
    return out[0, 0]


# speedup vs baseline: 3.2012x; 1.1298x over previous
import functools

import jax
import jax.numpy as jnp
from jax import lax
from jax.experimental import pallas as pl
from jax.experimental.pallas import tpu as pltpu
from jax.experimental.pallas import tpu_sc as plsc

_B, _NQ, _C, _NG = 32, 300, 80, 50
_NQP = 304
_ALPHA = 0.75
_W_VFL, _W_BBOX, _W_GIOU = 1.0, 5.0, 2.0
_L = 16
_BBLK = 8


def _sc_body(boxes_hbm, tgtb_hbm, sidx_hbm, lab_hbm,
             mask_hbm, iou_hbm, labf_hbm, pacc_hbm,
             boxes_v, tgtb_v, sidx_v, labi_v,
             mask_v, iou_v, labf_v, pacc_v):
    b = lax.axis_index("s") * 2 + lax.axis_index("c")
    pltpu.sync_copy(boxes_hbm.at[b], boxes_v)
    pltpu.sync_copy(tgtb_hbm.at[b], tgtb_v)
    pltpu.sync_copy(sidx_hbm, sidx_v.at[pl.ds(0, _B * _NG)])
    pltpu.sync_copy(lab_hbm, labi_v.at[pl.ds(0, _B * _NG)])

    zeros = jnp.zeros((_L,), jnp.float32)
    ones = jnp.ones((_L,), jnp.float32)
    for i in range(_NQP // _L):
        sl = pl.ds(i * _L, _L)
        mask_v[sl] = zeros
        iou_v[sl] = zeros
        labf_v[sl] = zeros

    lane = lax.iota(jnp.int32, _L)
    base = b * _NG
    acc = zeros
    for g in range(64 // _L):
        qi = sidx_v[pl.ds(base + g * _L, _L)]
        li = labi_v[pl.ds(base + g * _L, _L)]
        gl = lane + (g * _L)
        valid = gl < _NG
        qi = jnp.minimum(jnp.maximum(qi, 0), _NQ - 1)
        li = jnp.minimum(jnp.maximum(li, 0), _C - 1)
        qb = qi * 4
        scx = plsc.load_gather(boxes_v, [qb])
        scy = plsc.load_gather(boxes_v, [qb + 1])
        sw = plsc.load_gather(boxes_v, [qb + 2])
        sh = plsc.load_gather(boxes_v, [qb + 3])
        tq = jnp.minimum(gl, _NG - 1) * 4
        tcx = plsc.load_gather(tgtb_v, [tq])
        tcy = plsc.load_gather(tgtb_v, [tq + 1])
        tw = plsc.load_gather(tgtb_v, [tq + 2])
        th = plsc.load_gather(tgtb_v, [tq + 3])
        sx1 = scx - 0.5 * sw
        sy1 = scy - 0.5 * sh
        sx2 = scx + 0.5 * sw
        sy2 = scy + 0.5 * sh
        tx1 = tcx - 0.5 * tw
        ty1 = tcy - 0.5 * th
        tx2 = tcx + 0.5 * tw
        ty2 = tcy + 0.5 * th
        a1 = (sx2 - sx1) * (sy2 - sy1)
        a2 = (tx2 - tx1) * (ty2 - ty1)
        iw = jnp.maximum(jnp.minimum(sx2, tx2) - jnp.maximum(sx1, tx1), 0.0)
        ih = jnp.maximum(jnp.minimum(sy2, ty2) - jnp.maximum(sy1, ty1), 0.0)
        inter = iw * ih
        union = a1 + a2 - inter
        iou = jnp.maximum(inter / union, 0.0)
        ew = jnp.maximum(sx2, tx2) - jnp.minimum(sx1, tx1)
        eh = jnp.maximum(sy2, ty2) - jnp.minimum(sy1, ty1)
        enc = ew * eh
        giou = iou - (enc - union) / enc
        l1 = (jnp.abs(scx - tcx) + jnp.abs(scy - tcy)
              + jnp.abs(sw - tw) + jnp.abs(sh - th))
        pair = _W_BBOX * l1 + _W_GIOU * (1.0 - giou)
        acc = acc + jnp.where(valid, pair, zeros)
        plsc.store_scatter(mask_v, [qi], ones, mask=valid)
        plsc.store_scatter(iou_v, [qi], iou, mask=valid)
        plsc.store_scatter(labf_v, [qi], li.astype(jnp.float32), mask=valid)
    pacc_v[pl.ds(0, _L)] = acc

    pltpu.sync_copy(mask_v, mask_hbm.at[b])
    pltpu.sync_copy(iou_v, iou_hbm.at[b])
    pltpu.sync_copy(labf_v, labf_hbm.at[b])
    pltpu.sync_copy(pacc_v, pacc_hbm.at[b])


@functools.cache
def _get_sc_call():
    return pl.kernel(
        _sc_body,
        out_type=(
            jax.ShapeDtypeStruct((_B, _NQP), jnp.float32),
            jax.ShapeDtypeStruct((_B, _NQP), jnp.float32),
            jax.ShapeDtypeStruct((_B, _NQP), jnp.float32),
            jax.ShapeDtypeStruct((_B, _L), jnp.float32),
        ),
        mesh=plsc.VectorSubcoreMesh(core_axis_name="c", subcore_axis_name="s"),
        compiler_params=pltpu.CompilerParams(needs_layout_passes=False),
        scratch_types=[
            pltpu.VMEM((_NQ * 4,), jnp.float32),
            pltpu.VMEM((_NG * 4,), jnp.float32),
            pltpu.VMEM((_B * _NG + _L,), jnp.int32),
            pltpu.VMEM((_B * _NG + _L,), jnp.int32),
            pltpu.VMEM((_NQP,), jnp.float32),
            pltpu.VMEM((_NQP,), jnp.float32),
            pltpu.VMEM((_NQP,), jnp.float32),
            pltpu.VMEM((_L,), jnp.float32),
        ],
    )


def _tc_body(x_ref, mask_ref, iou_ref, labf_ref, pacc_ref, out_ref):
    j = pl.program_id(0)
    x = x_ref[...]
    t = jnp.exp(-jnp.abs(x))
    sig = jnp.where(x >= 0.0, 1.0, t) / (1.0 + t)
    sp = jnp.maximum(x, 0.0) + jnp.log1p(t)
    neg = _ALPHA * (sig * sig) * sp
    m = mask_ref[...][:, :_NQ]
    iou = iou_ref[...][:, :_NQ][:, :, None]
    labf = labf_ref[...][:, :_NQ][:, :, None]
    c_iota = lax.broadcasted_iota(jnp.int32, (_BBLK, _NQ, _C), 2)
    onehot = (c_iota == labf.astype(jnp.int32)).astype(jnp.float32)
    contrib = jnp.sum(neg * (1.0 - m)[:, :, None]
                      + onehot * (iou * (sp - iou * x)))
    total = _W_VFL * contrib + jnp.where(j == 0, jnp.sum(pacc_ref[...]), 0.0)
    prev = jnp.where(j == 0, 0.0, out_ref[0, 0])
    out_ref[0, 0] = prev + total * (1.0 / float(_B * _NG))


@functools.cache
def _get_tc_call():
    nblk = _B // _BBLK
    return pl.pallas_call(
        _tc_body,
        grid=(nblk,),
        in_specs=[
            pl.BlockSpec((_BBLK, _NQ, _C), lambda j: (j, 0, 0)),
            pl.BlockSpec((_BBLK, _NQP), lambda j: (j, 0)),
            pl.BlockSpec((_BBLK, _NQP), lambda j: (j, 0)),
            pl.BlockSpec((_BBLK, _NQP), lambda j: (j, 0)),
            pl.BlockSpec((_B, _L), lambda j: (0, 0)),
        ],
        out_shape=jax.ShapeDtypeStruct((1, 1), jnp.float32),
        out_specs=pl.BlockSpec(memory_space=pltpu.SMEM),
    )


def kernel(pred_logits, pred_boxes, tgt_boxes, tgt_labels, src_idx):
    boxes_flat = pred_boxes.reshape(_B, _NQ * 4)
    tgtb_flat = tgt_boxes.reshape(_B, _NG * 4)
    sidx_flat = src_idx.astype(jnp.int32).reshape(_B * _NG)
    lab_flat = tgt_labels.astype(jnp.int32).reshape(_B * _NG)
    mask, iou, labf, pacc = _get_sc_call()(
        boxes_flat, tgtb_flat, sidx_flat, lab_flat)
    out = _get_tc_call()(pred_logits, mask, iou, labf, pacc)
    return out[0, 0]


# speedup vs baseline: 3.2611x; 1.0187x over previous
import functools

import jax
import jax.numpy as jnp
from jax import lax
from jax.experimental import pallas as pl
from jax.experimental.pallas import tpu as pltpu
from jax.experimental.pallas import tpu_sc as plsc

_B, _NQ, _C, _NG = 32, 300, 80, 50
_NQP = 304
_ALPHA = 0.75
_W_VFL, _W_BBOX, _W_GIOU = 1.0, 5.0, 2.0
_L = 16
_BBLK = 8


def _sc_body(boxes_hbm, tgtb_hbm, sidx_hbm, lab_hbm,
             mask_hbm, iou_hbm, labf_hbm, pacc_hbm,
             boxes_v, tgtb_v, sidx_v, labi_v,
             mask_v, iou_v, labf_v, pacc_v, sem):
    b = lax.axis_index("s") * 2 + lax.axis_index("c")
    cp0 = pltpu.async_copy(boxes_hbm.at[b], boxes_v, sem)
    cp1 = pltpu.async_copy(tgtb_hbm.at[b], tgtb_v, sem)
    cp2 = pltpu.async_copy(sidx_hbm, sidx_v.at[pl.ds(0, _B * _NG)], sem)
    cp3 = pltpu.async_copy(lab_hbm, labi_v.at[pl.ds(0, _B * _NG)], sem)

    zeros = jnp.zeros((_L,), jnp.float32)
    ones = jnp.ones((_L,), jnp.float32)
    for i in range(_NQP // _L):
        sl = pl.ds(i * _L, _L)
        mask_v[sl] = zeros
        iou_v[sl] = zeros
        labf_v[sl] = zeros
    cp0.wait()
    cp1.wait()
    cp2.wait()
    cp3.wait()

    lane = lax.iota(jnp.int32, _L)
    base = b * _NG
    acc = zeros
    for g in range(64 // _L):
        qi = sidx_v[pl.ds(base + g * _L, _L)]
        li = labi_v[pl.ds(base + g * _L, _L)]
        gl = lane + (g * _L)
        valid = gl < _NG
        qi = jnp.minimum(jnp.maximum(qi, 0), _NQ - 1)
        li = jnp.minimum(jnp.maximum(li, 0), _C - 1)
        qb = qi * 4
        scx = plsc.load_gather(boxes_v, [qb])
        scy = plsc.load_gather(boxes_v, [qb + 1])
        sw = plsc.load_gather(boxes_v, [qb + 2])
        sh = plsc.load_gather(boxes_v, [qb + 3])
        tq = jnp.minimum(gl, _NG - 1) * 4
        tcx = plsc.load_gather(tgtb_v, [tq])
        tcy = plsc.load_gather(tgtb_v, [tq + 1])
        tw = plsc.load_gather(tgtb_v, [tq + 2])
        th = plsc.load_gather(tgtb_v, [tq + 3])
        sx1 = scx - 0.5 * sw
        sy1 = scy - 0.5 * sh
        sx2 = scx + 0.5 * sw
        sy2 = scy + 0.5 * sh
        tx1 = tcx - 0.5 * tw
        ty1 = tcy - 0.5 * th
        tx2 = tcx + 0.5 * tw
        ty2 = tcy + 0.5 * th
        a1 = (sx2 - sx1) * (sy2 - sy1)
        a2 = (tx2 - tx1) * (ty2 - ty1)
        iw = jnp.maximum(jnp.minimum(sx2, tx2) - jnp.maximum(sx1, tx1), 0.0)
        ih = jnp.maximum(jnp.minimum(sy2, ty2) - jnp.maximum(sy1, ty1), 0.0)
        inter = iw * ih
        union = a1 + a2 - inter
        iou = jnp.maximum(inter / union, 0.0)
        ew = jnp.maximum(sx2, tx2) - jnp.minimum(sx1, tx1)
        eh = jnp.maximum(sy2, ty2) - jnp.minimum(sy1, ty1)
        enc = ew * eh
        giou = iou - (enc - union) / enc
        l1 = (jnp.abs(scx - tcx) + jnp.abs(scy - tcy)
              + jnp.abs(sw - tw) + jnp.abs(sh - th))
        pair = _W_BBOX * l1 + _W_GIOU * (1.0 - giou)
        acc = acc + jnp.where(valid, pair, zeros)
        plsc.store_scatter(mask_v, [qi], ones, mask=valid)
        plsc.store_scatter(iou_v, [qi], iou, mask=valid)
        plsc.store_scatter(labf_v, [qi], li.astype(jnp.float32), mask=valid)
    pacc_v[pl.ds(0, _L)] = acc

    co0 = pltpu.async_copy(mask_v, mask_hbm.at[b], sem)
    co1 = pltpu.async_copy(iou_v, iou_hbm.at[b], sem)
    co2 = pltpu.async_copy(labf_v, labf_hbm.at[b], sem)
    co3 = pltpu.async_copy(pacc_v, pacc_hbm.at[b], sem)
    co0.wait()
    co1.wait()
    co2.wait()
    co3.wait()


@functools.cache
def _get_sc_call():
    return pl.kernel(
        _sc_body,
        out_type=(
            jax.ShapeDtypeStruct((_B, _NQP), jnp.float32),
            jax.ShapeDtypeStruct((_B, _NQP), jnp.float32),
            jax.ShapeDtypeStruct((_B, _NQP), jnp.float32),
            jax.ShapeDtypeStruct((_B, _L), jnp.float32),
        ),
        mesh=plsc.VectorSubcoreMesh(core_axis_name="c", subcore_axis_name="s"),
        compiler_params=pltpu.CompilerParams(needs_layout_passes=False),
        scratch_types=[
            pltpu.VMEM((_NQ * 4,), jnp.float32),
            pltpu.VMEM((_NG * 4,), jnp.float32),
            pltpu.VMEM((_B * _NG + _L,), jnp.int32),
            pltpu.VMEM((_B * _NG + _L,), jnp.int32),
            pltpu.VMEM((_NQP,), jnp.float32),
            pltpu.VMEM((_NQP,), jnp.float32),
            pltpu.VMEM((_NQP,), jnp.float32),
            pltpu.VMEM((_L,), jnp.float32),
            pltpu.SemaphoreType.DMA,
        ],
    )


def _tc_body(x_ref, mask_ref, iou_ref, labf_ref, pacc_ref, out_ref):
    j = pl.program_id(0)
    x = x_ref[...]
    t = jnp.exp(-jnp.abs(x))
    sig = jnp.where(x >= 0.0, 1.0, t) / (1.0 + t)
    sp = jnp.maximum(x, 0.0) + jnp.log1p(t)
    neg = _ALPHA * (sig * sig) * sp
    m = mask_ref[...][:, :_NQ]
    iou = iou_ref[...][:, :_NQ][:, :, None]
    labf = labf_ref[...][:, :_NQ][:, :, None]
    c_iota = lax.broadcasted_iota(jnp.int32, (_BBLK, _NQ, _C), 2)
    onehot = (c_iota == labf.astype(jnp.int32)).astype(jnp.float32)
    contrib = jnp.sum(neg * (1.0 - m)[:, :, None]
                      + onehot * (iou * (sp - iou * x)))
    total = _W_VFL * contrib + jnp.where(j == 0, jnp.sum(pacc_ref[...]), 0.0)
    prev = jnp.where(j == 0, 0.0, out_ref[0, 0])
    out_ref[0, 0] = prev + total * (1.0 / float(_B * _NG))


@functools.cache
def _get_tc_call():
    nblk = _B // _BBLK
    return pl.pallas_call(
        _tc_body,
        grid=(nblk,),
        in_specs=[
            pl.BlockSpec((_BBLK, _NQ, _C), lambda j: (j, 0, 0)),
            pl.BlockSpec((_BBLK, _NQP), lambda j: (j, 0)),
            pl.BlockSpec((_BBLK, _NQP), lambda j: (j, 0)),
            pl.BlockSpec((_BBLK, _NQP), lambda j: (j, 0)),
            pl.BlockSpec((_B, _L), lambda j: (0, 0)),
        ],
        out_shape=jax.ShapeDtypeStruct((1, 1), jnp.float32),
        out_specs=pl.BlockSpec(memory_space=pltpu.SMEM),
    )


def kernel(pred_logits, pred_boxes, tgt_boxes, tgt_labels, src_idx):
    boxes_flat = pred_boxes.reshape(_B, _NQ * 4)
    tgtb_flat = tgt_boxes.reshape(_B, _NG * 4)
    sidx_flat = src_idx.astype(jnp.int32).reshape(_B * _NG)
    lab_flat = tgt_labels.astype(jnp.int32).reshape(_B * _NG)
    mask, iou, labf, pacc = _get_sc_call()(
        boxes_flat, tgtb_flat, sidx_flat, lab_flat)
    out = _get_tc_call()(pred_logits, mask, iou, labf, pacc)
    return out[0, 0]


# speedup vs baseline: 3.4012x; 1.0430x over previous
import functools

import jax
import jax.numpy as jnp
from jax import lax
from jax.experimental import pallas as pl
from jax.experimental.pallas import tpu as pltpu
from jax.experimental.pallas import tpu_sc as plsc

_B, _NQ, _C, _NG = 32, 300, 80, 50
_ALPHA = 0.75
_W_VFL, _W_BBOX, _W_GIOU = 1.0, 5.0, 2.0
_L = 16
_BBLK = 8

_OB, _OT, _OS, _OLAB, _IN_W = 0, 1200, 1400, 1456, 1520
_OM, _OI, _OL, _OP, _OUT_W = 0, 384, 768, 1152, 1280


def _sc_body(in_hbm, out_hbm, in_v, out_v, sem):
    b = lax.axis_index("s") * 2 + lax.axis_index("c")
    cp = pltpu.async_copy(in_hbm.at[b], in_v, sem)
    zeros = jnp.zeros((_L,), jnp.float32)
    ones = jnp.ones((_L,), jnp.float32)
    for i in range(_OUT_W // _L):
        out_v[pl.ds(i * _L, _L)] = zeros
    cp.wait()

    lane = lax.iota(jnp.int32, _L)
    acc = zeros
    for g in range(64 // _L):
        qi = in_v[pl.ds(_OS + g * _L, _L)].astype(jnp.int32)
        li = in_v[pl.ds(_OLAB + g * _L, _L)]
        gl = lane + (g * _L)
        valid = gl < _NG
        qi = jnp.minimum(jnp.maximum(qi, 0), _NQ - 1)
        qb = qi * 4
        scx = plsc.load_gather(in_v, [qb])
        scy = plsc.load_gather(in_v, [qb + 1])
        sw = plsc.load_gather(in_v, [qb + 2])
        sh = plsc.load_gather(in_v, [qb + 3])
        tq = _OT + jnp.minimum(gl, _NG - 1) * 4
        tcx = plsc.load_gather(in_v, [tq])
        tcy = plsc.load_gather(in_v, [tq + 1])
        tw = plsc.load_gather(in_v, [tq + 2])
        th = plsc.load_gather(in_v, [tq + 3])
        sx1 = scx - 0.5 * sw
        sy1 = scy - 0.5 * sh
        sx2 = scx + 0.5 * sw
        sy2 = scy + 0.5 * sh
        tx1 = tcx - 0.5 * tw
        ty1 = tcy - 0.5 * th
        tx2 = tcx + 0.5 * tw
        ty2 = tcy + 0.5 * th
        a1 = (sx2 - sx1) * (sy2 - sy1)
        a2 = (tx2 - tx1) * (ty2 - ty1)
        iw = jnp.maximum(jnp.minimum(sx2, tx2) - jnp.maximum(sx1, tx1), 0.0)
        ih = jnp.maximum(jnp.minimum(sy2, ty2) - jnp.maximum(sy1, ty1), 0.0)
        inter = iw * ih
        union = a1 + a2 - inter
        iou = jnp.maximum(inter / union, 0.0)
        ew = jnp.maximum(sx2, tx2) - jnp.minimum(sx1, tx1)
        eh = jnp.maximum(sy2, ty2) - jnp.minimum(sy1, ty1)
        enc = ew * eh
        giou = iou - (enc - union) / enc
        l1 = (jnp.abs(scx - tcx) + jnp.abs(scy - tcy)
              + jnp.abs(sw - tw) + jnp.abs(sh - th))
        pair = _W_BBOX * l1 + _W_GIOU * (1.0 - giou)
        acc = acc + jnp.where(valid, pair, zeros)
        plsc.store_scatter(out_v, [qi + _OM], ones, mask=valid)
        plsc.store_scatter(out_v, [qi + _OI], iou, mask=valid)
        plsc.store_scatter(out_v, [qi + _OL], li, mask=valid)
    out_v[pl.ds(_OP, _L)] = acc

    pltpu.async_copy(out_v, out_hbm.at[b], sem).wait()


@functools.cache
def _get_sc_call():
    return pl.kernel(
        _sc_body,
        out_type=jax.ShapeDtypeStruct((_B, _OUT_W), jnp.float32),
        mesh=plsc.VectorSubcoreMesh(core_axis_name="c", subcore_axis_name="s"),
        compiler_params=pltpu.CompilerParams(needs_layout_passes=False),
        scratch_types=[
            pltpu.VMEM((_IN_W,), jnp.float32),
            pltpu.VMEM((_OUT_W,), jnp.float32),
            pltpu.SemaphoreType.DMA,
        ],
    )


def _tc_body(x_ref, scat_ref, out_ref):
    j = pl.program_id(0)
    x = x_ref[...]
    scat = scat_ref[...]
    t = jnp.exp(-jnp.abs(x))
    sig = jnp.where(x >= 0.0, 1.0, t) / (1.0 + t)
    sp = jnp.maximum(x, 0.0) + jnp.log1p(t)
    m = scat[:, _OM:_OM + _NQ]
    iou = scat[:, _OI:_OI + _NQ][:, :, None]
    labf = scat[:, _OL:_OL + _NQ][:, :, None]
    w = (_ALPHA * (1.0 - m))[:, :, None] * (sig * sig)
    c_iota = lax.broadcasted_iota(jnp.int32, (_BBLK, _NQ, _C), 2)
    iou_oh = jnp.where(c_iota == labf.astype(jnp.int32), iou, 0.0)
    contrib = jnp.sum(sp * (w + iou_oh) - (iou_oh * iou) * x)
    total = _W_VFL * contrib + jnp.sum(scat[:, _OP:_OP + _L])
    prev = jnp.where(j == 0, 0.0, out_ref[0, 0])
    out_ref[0, 0] = prev + total * (1.0 / float(_B * _NG))


@functools.cache
def _get_tc_call():
    nblk = _B // _BBLK
    return pl.pallas_call(
        _tc_body,
        grid=(nblk,),
        in_specs=[
            pl.BlockSpec((_BBLK, _NQ, _C), lambda j: (j, 0, 0)),
            pl.BlockSpec((_BBLK, _OUT_W), lambda j: (j, 0)),
        ],
        out_shape=jax.ShapeDtypeStruct((1, 1), jnp.float32),
        out_specs=pl.BlockSpec(memory_space=pltpu.SMEM),
    )


def kernel(pred_logits, pred_boxes, tgt_boxes, tgt_labels, src_idx):
    packed = jnp.concatenate([
        pred_boxes.reshape(_B, _NQ * 4),
        tgt_boxes.reshape(_B, _NG * 4),
        jnp.pad(src_idx.astype(jnp.float32), ((0, 0), (0, 56 - _NG))),
        jnp.pad(tgt_labels.astype(jnp.float32), ((0, 0), (0, 64 - _NG))),
    ], axis=1)
    scat = _get_sc_call()(packed)
    out = _get_tc_call()(pred_logits, scat)
    return out[0, 0]


# speedup vs baseline: 3.9891x; 1.1729x over previous
import functools

import jax
import jax.numpy as jnp
from jax import lax
from jax.experimental import pallas as pl
from jax.experimental.pallas import tpu as pltpu
from jax.experimental.pallas import tpu_sc as plsc

_B, _NQ, _C, _NG = 32, 300, 80, 50
_ALPHA = 0.75
_W_VFL, _W_BBOX, _W_GIOU = 1.0, 5.0, 2.0
_L = 16
_BBLK = 8

_OT, _OS, _OLAB, _IN_W = 1200, 1400, 1456, 1520


def _pack_body(bx_hbm, tg_hbm, si_hbm, la_hbm, out_ref, bxv, tgv, siv, lav, sem):
    c0 = pltpu.async_copy(bx_hbm, bxv, sem)
    c1 = pltpu.async_copy(tg_hbm, tgv, sem)
    c2 = pltpu.async_copy(si_hbm, siv, sem)
    c3 = pltpu.async_copy(la_hbm, lav, sem)
    c0.wait()
    c1.wait()
    c2.wait()
    c3.wait()
    for k in range(4):
        out_ref[:, k * _NQ:(k + 1) * _NQ] = bxv[:, k, :]
        out_ref[:, _OT + k * _NG:_OT + (k + 1) * _NG] = tgv[:, k, :]
    out_ref[:, _OS:_OS + _NG] = siv[...].astype(jnp.float32)
    out_ref[:, _OS + _NG:_OLAB] = jnp.zeros((_B, 56 - _NG), jnp.float32)
    out_ref[:, _OLAB:_OLAB + _NG] = lav[...].astype(jnp.float32)
    out_ref[:, _OLAB + _NG:_IN_W] = jnp.zeros((_B, 64 - _NG), jnp.float32)


@functools.cache
def _get_pack_call():
    hbm = pl.BlockSpec(memory_space=pltpu.MemorySpace.HBM)
    return pl.pallas_call(
        _pack_body,
        in_specs=[hbm, hbm, hbm, hbm],
        out_shape=jax.ShapeDtypeStruct((_B, _IN_W), jnp.float32),
        scratch_shapes=[
            pltpu.VMEM((_B, 4, _NQ), jnp.float32),
            pltpu.VMEM((_B, 4, _NG), jnp.float32),
            pltpu.VMEM((_B, _NG), jnp.int32),
            pltpu.VMEM((_B, _NG), jnp.int32),
            pltpu.SemaphoreType.DMA,
        ],
    )
_OM, _OI, _OL, _OP, _OUT_W = 0, 384, 768, 1152, 1280


def _sc_body(in_hbm, out_hbm, in_v, out_v, sem):
    b = lax.axis_index("s") * 2 + lax.axis_index("c")
    cp = pltpu.async_copy(in_hbm.at[b], in_v, sem)
    zeros = jnp.zeros((_L,), jnp.float32)
    ones = jnp.ones((_L,), jnp.float32)
    for i in range(_OUT_W // _L):
        out_v[pl.ds(i * _L, _L)] = zeros
    cp.wait()

    lane = lax.iota(jnp.int32, _L)
    acc = zeros
    for g in range(64 // _L):
        qi = in_v[pl.ds(_OS + g * _L, _L)].astype(jnp.int32)
        li = in_v[pl.ds(_OLAB + g * _L, _L)]
        gl = lane + (g * _L)
        valid = gl < _NG
        qi = jnp.minimum(jnp.maximum(qi, 0), _NQ - 1)
        scx = plsc.load_gather(in_v, [qi])
        scy = plsc.load_gather(in_v, [qi + _NQ])
        sw = plsc.load_gather(in_v, [qi + 2 * _NQ])
        sh = plsc.load_gather(in_v, [qi + 3 * _NQ])
        tq = _OT + jnp.minimum(gl, _NG - 1)
        tcx = plsc.load_gather(in_v, [tq])
        tcy = plsc.load_gather(in_v, [tq + _NG])
        tw = plsc.load_gather(in_v, [tq + 2 * _NG])
        th = plsc.load_gather(in_v, [tq + 3 * _NG])
        sx1 = scx - 0.5 * sw
        sy1 = scy - 0.5 * sh
        sx2 = scx + 0.5 * sw
        sy2 = scy + 0.5 * sh
        tx1 = tcx - 0.5 * tw
        ty1 = tcy - 0.5 * th
        tx2 = tcx + 0.5 * tw
        ty2 = tcy + 0.5 * th
        a1 = (sx2 - sx1) * (sy2 - sy1)
        a2 = (tx2 - tx1) * (ty2 - ty1)
        iw = jnp.maximum(jnp.minimum(sx2, tx2) - jnp.maximum(sx1, tx1), 0.0)
        ih = jnp.maximum(jnp.minimum(sy2, ty2) - jnp.maximum(sy1, ty1), 0.0)
        inter = iw * ih
        union = a1 + a2 - inter
        iou = jnp.maximum(inter / union, 0.0)
        ew = jnp.maximum(sx2, tx2) - jnp.minimum(sx1, tx1)
        eh = jnp.maximum(sy2, ty2) - jnp.minimum(sy1, ty1)
        enc = ew * eh
        giou = iou - (enc - union) / enc
        l1 = (jnp.abs(scx - tcx) + jnp.abs(scy - tcy)
              + jnp.abs(sw - tw) + jnp.abs(sh - th))
        pair = _W_BBOX * l1 + _W_GIOU * (1.0 - giou)
        acc = acc + jnp.where(valid, pair, zeros)
        plsc.store_scatter(out_v, [qi + _OM], ones, mask=valid)
        plsc.store_scatter(out_v, [qi + _OI], iou, mask=valid)
        plsc.store_scatter(out_v, [qi + _OL], li, mask=valid)
    out_v[pl.ds(_OP, _L)] = acc

    pltpu.async_copy(out_v, out_hbm.at[b], sem).wait()


@functools.cache
def _get_sc_call():
    return pl.kernel(
        _sc_body,
        out_type=jax.ShapeDtypeStruct((_B, _OUT_W), jnp.float32),
        mesh=plsc.VectorSubcoreMesh(core_axis_name="c", subcore_axis_name="s"),
        compiler_params=pltpu.CompilerParams(needs_layout_passes=False),
        scratch_types=[
            pltpu.VMEM((_IN_W,), jnp.float32),
            pltpu.VMEM((_OUT_W,), jnp.float32),
            pltpu.SemaphoreType.DMA,
        ],
    )


def _tc_body(x_hbm, scat_ref, out_ref, xbuf, sems):
    j = pl.program_id(0)
    nblk = _B // _BBLK

    @pl.when(j == 0)
    def _():
        pltpu.make_async_copy(
            x_hbm.at[pl.ds(0, _BBLK)], xbuf.at[0], sems.at[0]).start()

    @pl.when(j + 1 < nblk)
    def _():
        pltpu.make_async_copy(
            x_hbm.at[pl.ds((j + 1) * _BBLK, _BBLK)],
            xbuf.at[(j + 1) % 2], sems.at[(j + 1) % 2]).start()

    pltpu.make_async_copy(
        x_hbm.at[pl.ds(j * _BBLK, _BBLK)], xbuf.at[j % 2],
        sems.at[j % 2]).wait()
    x = xbuf[j % 2]
    scat = scat_ref[...]
    t = jnp.exp(-jnp.abs(x))
    sig = jnp.where(x >= 0.0, 1.0, t) / (1.0 + t)
    sp = jnp.maximum(x, 0.0) + jnp.log1p(t)
    m = scat[:, _OM:_OM + _NQ]
    iou = scat[:, _OI:_OI + _NQ][:, None, :]
    labf = scat[:, _OL:_OL + _NQ][:, None, :]
    w = (_ALPHA * (1.0 - m))[:, None, :] * (sig * sig)
    c_iota = lax.broadcasted_iota(jnp.int32, (_BBLK, _C, _NQ), 1)
    iou_oh = jnp.where(c_iota == labf.astype(jnp.int32), iou, 0.0)
    contrib = jnp.sum(sp * (w + iou_oh) - (iou_oh * iou) * x)
    total = _W_VFL * contrib + jnp.sum(scat[:, _OP:_OP + _L])
    prev = jnp.where(j == 0, 0.0, out_ref[0, 0])
    out_ref[0, 0] = prev + total * (1.0 / float(_B * _NG))


@functools.cache
def _get_tc_call():
    nblk = _B // _BBLK
    return pl.pallas_call(
        _tc_body,
        grid=(nblk,),
        in_specs=[
            pl.BlockSpec(memory_space=pltpu.MemorySpace.HBM),
            pl.BlockSpec((_BBLK, _OUT_W), lambda j: (j, 0)),
        ],
        out_shape=jax.ShapeDtypeStruct((1, 1), jnp.float32),
        out_specs=pl.BlockSpec(memory_space=pltpu.SMEM),
        scratch_shapes=[
            pltpu.VMEM((2, _BBLK, _C, _NQ), jnp.float32),
            pltpu.SemaphoreType.DMA((2,)),
        ],
    )


def kernel(pred_logits, pred_boxes, tgt_boxes, tgt_labels, src_idx):
    pin = lambda a: pltpu.with_memory_space_constraint(a, pltpu.MemorySpace.HBM)
    packed = _get_pack_call()(
        pin(jnp.swapaxes(pred_boxes, 1, 2)),
        pin(jnp.swapaxes(tgt_boxes, 1, 2)),
        pin(src_idx.astype(jnp.int32)),
        pin(tgt_labels.astype(jnp.int32)),
    )
    scat = _get_sc_call()(packed)
    x_hbm = pltpu.with_memory_space_constraint(
        jnp.swapaxes(pred_logits, 1, 2), pltpu.MemorySpace.HBM)
    out = _get_tc_call()(x_hbm, scat)
    return out[0, 0]
